# baseline (speedup 1.0000x reference)
import functools

import jax
import jax.numpy as jnp
from jax import lax
from jax.experimental import pallas as pl
from jax.experimental.pallas import tpu as pltpu
from jax.experimental.pallas import tpu_sc as plsc

_B = 16384
_D = 32
_L = 16
_NW = 32
_BPW = _B // _NW
_V = 1000000
_RPW = 31360
_WIN = 512
_NWIN = 62
_SMAX = 999552
_NSTAGE = 128
_DUMP = _B
_SROWS = _B + _NSTAGE

_mesh = plsc.VectorSubcoreMesh(core_axis_name="c", subcore_axis_name="s")


@functools.partial(
    pl.kernel,
    out_type=(jax.ShapeDtypeStruct((_SROWS, 128), jnp.float32),
              jax.ShapeDtypeStruct((_SROWS, 128), jnp.float32)),
    mesh=_mesh,
    scratch_types=[
        pltpu.VMEM((_B,), jnp.int32),
        pltpu.VMEM((_B,), jnp.int32),
        pltpu.VMEM((2, _D, _WIN), jnp.float32),
        pltpu.VMEM((_NSTAGE, 128), jnp.float32),
        pltpu.VMEM((_NSTAGE,), jnp.int32),
        pltpu.SemaphoreType.DMA((2,)),
        pltpu.SemaphoreType.DMA,
    ],
    compiler_params=pltpu.CompilerParams(needs_layout_passes=False),
)
def _gmf_sweep(user_hbm, item_hbm, utab_hbm, itab_hbm, urows_hbm, irows_hbm,
               idx_v, ent, blk, srows, sidx, wsems, fsem):
    wid = lax.axis_index("s") * 2 + lax.axis_index("c")
    lo = wid * _RPW
    rlen = jnp.minimum(lo + _RPW, _V) - lo
    dio = lax.iota(jnp.int32, _L)
    dio_hi = dio + _L
    lane0 = dio == 0

    def win_start(j):
        return jnp.minimum(lo + j * _WIN, _SMAX)

    def fire(tab, j, slot):
        off = pl.multiple_of(win_start(j), 128)
        pltpu.async_copy(tab.at[:, pl.ds(off, _WIN)], blk.at[slot],
                         wsems.at[slot])

    def drain(tab, slot):
        pltpu.make_async_copy(
            tab.at[:, pl.ds(0, _WIN)], blk.at[slot], wsems.at[slot]).wait()

    def run_table(tab, src_idx_hbm, out_rows_hbm):
        pltpu.sync_copy(src_idx_hbm, idx_v)

        def cbody(v, ptr):
            voff = pl.multiple_of(v * _L, _L)
            iv = idx_v[pl.ds(voff, _L)]
            rel = iv - lo
            m = (rel >= 0) & (rel < rlen)
            packed = rel | ((v * _L + dio) << 15)
            plsc.store_compressed(ent.at[pl.ds(ptr, _L)], packed, mask=m)
            return ptr + plsc.all_reduce_population_count(m)[0]

        nent = lax.fori_loop(0, _B // _L, cbody, jnp.int32(0))
        nv = (nent + _L - 1) // _L

        def process(j, slot, k):
            s_rel = win_start(j) - lo

            def qbody(q, k):
                qoff = pl.multiple_of(q * _L, _L)
                ev = ent[pl.ds(qoff, _L)]
                rel_e = ev & 32767
                mm = ((q * _L + dio) < nent) & (rel_e >= s_rel) \
                    & (rel_e < s_rel + _WIN)

                def wcond(carry):
                    m, _ = carry
                    return jnp.any(m)

                def wbody(carry):
                    m, k = carry
                    lane = plsc.all_reduce_ffs(m)[0]
                    val = plsc.load_gather(
                        ent, [jnp.broadcast_to(q * _L + lane, (_L,))])[0]
                    l = (val & 32767) - s_rel
                    b = val >> 15
                    lv = jnp.broadcast_to(l, (_L,))
                    vlo = plsc.load_gather(blk.at[slot], [dio, lv])
                    vhi = plsc.load_gather(blk.at[slot], [dio_hi, lv])
                    srows[k, pl.ds(0, _L)] = vlo
                    srows[k, pl.ds(_L, _L)] = vhi
                    plsc.store_scatter(sidx, [jnp.broadcast_to(k, (_L,))],
                                       jnp.broadcast_to(b, (_L,)), mask=lane0)
                    k = k + 1

                    @pl.when(k == _NSTAGE)
                    def _():
                        pltpu.async_copy(srows, out_rows_hbm.at[sidx],
                                         fsem).wait()
                    k = lax.select(k == _NSTAGE, jnp.int32(0), k)
                    return m & (dio != lane), k

                _, k = lax.while_loop(wcond, wbody, (mm, k))
                return k

            return lax.fori_loop(0, nv, qbody, k)

        fire(tab, 0, 0)

        def wloop(jj, k):
            j0 = jj * 2
            fire(tab, j0 + 1, 1)
            drain(tab, 0)
            k = process(j0, 0, k)

            @pl.when(jj < _NWIN // 2 - 1)
            def _():
                fire(tab, j0 + 2, 0)
            drain(tab, 1)
            return process(j0 + 1, 1, k)

        k = lax.fori_loop(0, _NWIN // 2, wloop, jnp.int32(0))

        for q in range(_NSTAGE // _L):
            ids = dio + q * _L
            cur = sidx[pl.ds(q * _L, _L)]
            sidx[pl.ds(q * _L, _L)] = jnp.where(ids < k, cur, _DUMP)
        pltpu.async_copy(srows, out_rows_hbm.at[sidx], fsem).wait()

    run_table(utab_hbm, user_hbm, urows_hbm)
    run_table(itab_hbm, item_hbm, irows_hbm)


_CH = 128


@functools.partial(
    pl.kernel,
    out_type=jax.ShapeDtypeStruct((_B,), jnp.float32),
    mesh=_mesh,
    scratch_types=[
        pltpu.VMEM((_CH, 128), jnp.float32),
        pltpu.VMEM((_CH, 128), jnp.float32),
        pltpu.VMEM((_BPW,), jnp.float32),
        pltpu.VMEM((3 * _L,), jnp.float32),
    ],
    compiler_params=pltpu.CompilerParams(needs_layout_passes=False),
)
def _gmf_join(urows_hbm, irows_hbm, params_hbm, out_hbm,
              ublk, iblk, out_v, p_v):
    wid = lax.axis_index("s") * 2 + lax.axis_index("c")
    base = wid * _BPW
    pltpu.sync_copy(params_hbm, p_v)
    w_lo = p_v[pl.ds(0, _L)]
    w_hi = p_v[pl.ds(_L, _L)]
    bias = p_v[pl.ds(2 * _L, _L)][0]
    dio = lax.iota(jnp.int32, _L)
    last_lane = dio == (_L - 1)

    def chunk(ci, carry):
        cb = pl.multiple_of(base + ci * _CH, _CH)
        pltpu.sync_copy(urows_hbm.at[pl.ds(cb, _CH), :], ublk)
        pltpu.sync_copy(irows_hbm.at[pl.ds(cb, _CH), :], iblk)

        def bbody(bi, carry):
            u_lo = ublk[bi, pl.ds(0, _L)]
            u_hi = ublk[bi, pl.ds(_L, _L)]
            i_lo = iblk[bi, pl.ds(0, _L)]
            i_hi = iblk[bi, pl.ds(_L, _L)]
            sv = u_lo * i_lo * w_lo + u_hi * i_hi * w_hi
            total = plsc.cumsum(sv) + bias
            plsc.store_scatter(out_v, [jnp.broadcast_to(ci * _CH + bi, (_L,))],
                               total, mask=last_lane)
            return carry

        return lax.fori_loop(0, _CH, bbody, carry)

    lax.fori_loop(0, _BPW // _CH, chunk, 0)
    pltpu.sync_copy(out_v, out_hbm.at[pl.ds(base, _BPW)])


def kernel(user, item, user_table, item_table, W, b):
    params = jnp.concatenate(
        [W.reshape(-1), b.reshape(-1), jnp.zeros((15,), jnp.float32)])
    urows, irows = _gmf_sweep(user.astype(jnp.int32), item.astype(jnp.int32),
                              user_table.T, item_table.T)
    return _gmf_join(urows, irows, params)


# speedup vs baseline: 1.1564x; 1.1564x over previous
import functools

import jax
import jax.numpy as jnp
from jax import lax
from jax.experimental import pallas as pl
from jax.experimental.pallas import tpu as pltpu
from jax.experimental.pallas import tpu_sc as plsc

_B = 16384
_D = 32
_L = 16
_NW = 32
_BPW = _B // _NW
_V = 1000000
_RPW = 31360
_WIN = 512
_NWIN = 62
_SMAX = 999552
_NSTAGE = 128
_DUMP = _B
_SROWS = _B + _NSTAGE

_mesh = plsc.VectorSubcoreMesh(core_axis_name="c", subcore_axis_name="s")


@functools.partial(
    pl.kernel,
    out_type=(jax.ShapeDtypeStruct((_SROWS, 128), jnp.float32),
              jax.ShapeDtypeStruct((_SROWS, 128), jnp.float32)),
    mesh=_mesh,
    scratch_types=[
        pltpu.VMEM((_B,), jnp.int32),
        pltpu.VMEM((_B,), jnp.int32),
        pltpu.VMEM((_B,), jnp.int32),
        pltpu.VMEM((2, _D, _WIN), jnp.float32),
        pltpu.VMEM((_NSTAGE, 128), jnp.float32),
        pltpu.VMEM((_NSTAGE,), jnp.int32),
        pltpu.SemaphoreType.DMA((2,)),
        pltpu.SemaphoreType.DMA,
    ],
    compiler_params=pltpu.CompilerParams(needs_layout_passes=False),
)
def _gmf_sweep(user_hbm, item_hbm, utab_hbm, itab_hbm, urows_hbm, irows_hbm,
               idx_v, ent, ent2, blk, srows, sidx, wsems, fsem):
    wid = lax.axis_index("s") * 2 + lax.axis_index("c")
    lo = wid * _RPW
    rlen = jnp.minimum(lo + _RPW, _V) - lo
    dio = lax.iota(jnp.int32, _L)
    dio_hi = dio + _L
    lane0 = dio == 0

    def win_start(j):
        return jnp.minimum(lo + j * _WIN, _SMAX)

    def fire(tab, j, slot):
        off = pl.multiple_of(win_start(j), 128)
        pltpu.async_copy(tab.at[:, pl.ds(off, _WIN)], blk.at[slot],
                         wsems.at[slot])

    def drain(tab, slot):
        pltpu.make_async_copy(
            tab.at[:, pl.ds(0, _WIN)], blk.at[slot], wsems.at[slot]).wait()

    def run_table(tab, src_idx_hbm, out_rows_hbm):
        pltpu.sync_copy(src_idx_hbm, idx_v)

        def cbody(v, ptr):
            voff = pl.multiple_of(v * _L, _L)
            iv = idx_v[pl.ds(voff, _L)]
            rel = iv - lo
            m = (rel >= 0) & (rel < rlen)
            packed = rel | ((v * _L + dio) << 15)
            plsc.store_compressed(ent.at[pl.ds(ptr, _L)], packed, mask=m)
            return ptr + plsc.all_reduce_population_count(m)[0]

        nent = lax.fori_loop(0, _B // _L, cbody, jnp.int32(0))
        nv = (nent + _L - 1) // _L

        def bucket(j, ptr2):
            s_rel = win_start(j) - lo

            def qbody(q, ptr2):
                qoff = pl.multiple_of(q * _L, _L)
                ev = ent[pl.ds(qoff, _L)]
                rel_e = ev & 32767
                mm = ((q * _L + dio) < nent) & (rel_e >= s_rel) \
                    & (rel_e < s_rel + _WIN)
                plsc.store_compressed(ent2.at[pl.ds(ptr2, _L)], ev, mask=mm)
                return ptr2 + plsc.all_reduce_population_count(mm)[0]

            return lax.fori_loop(0, nv, qbody, ptr2)

        def process(j, slot, p0, p1, k):
            s_rel = win_start(j) - lo

            def ebody(e, k):
                val = plsc.load_gather(
                    ent2, [jnp.broadcast_to(e, (_L,))])[0]
                l = (val & 32767) - s_rel
                b = val >> 15
                lv = jnp.broadcast_to(l, (_L,))
                vlo = plsc.load_gather(blk.at[slot], [dio, lv])
                vhi = plsc.load_gather(blk.at[slot], [dio_hi, lv])
                srows[k, pl.ds(0, _L)] = vlo
                srows[k, pl.ds(_L, _L)] = vhi
                plsc.store_scatter(sidx, [jnp.broadcast_to(k, (_L,))],
                                   jnp.broadcast_to(b, (_L,)), mask=lane0)
                k = k + 1

                @pl.when(k == _NSTAGE)
                def _():
                    pltpu.async_copy(srows, out_rows_hbm.at[sidx],
                                     fsem).wait()
                return lax.select(k == _NSTAGE, jnp.int32(0), k)

            return lax.fori_loop(p0, p1, ebody, k)

        fire(tab, 0, 0)

        def wloop(jj, carry):
            k, ptr2 = carry
            j0 = jj * 2
            fire(tab, j0 + 1, 1)
            p1 = bucket(j0, ptr2)
            drain(tab, 0)
            k = process(j0, 0, ptr2, p1, k)

            @pl.when(jj < _NWIN // 2 - 1)
            def _():
                fire(tab, j0 + 2, 0)
            p2 = bucket(j0 + 1, p1)
            drain(tab, 1)
            k = process(j0 + 1, 1, p1, p2, k)
            return k, p2

        k, _ = lax.fori_loop(0, _NWIN // 2, wloop,
                             (jnp.int32(0), jnp.int32(0)))

        for q in range(_NSTAGE // _L):
            ids = dio + q * _L
            cur = sidx[pl.ds(q * _L, _L)]
            sidx[pl.ds(q * _L, _L)] = jnp.where(ids < k, cur, _DUMP)
        pltpu.async_copy(srows, out_rows_hbm.at[sidx], fsem).wait()

    run_table(utab_hbm, user_hbm, urows_hbm)
    run_table(itab_hbm, item_hbm, irows_hbm)


_CH = 128


@functools.partial(
    pl.kernel,
    out_type=jax.ShapeDtypeStruct((_B,), jnp.float32),
    mesh=_mesh,
    scratch_types=[
        pltpu.VMEM((_CH, 128), jnp.float32),
        pltpu.VMEM((_CH, 128), jnp.float32),
        pltpu.VMEM((_BPW,), jnp.float32),
        pltpu.VMEM((3 * _L,), jnp.float32),
    ],
    compiler_params=pltpu.CompilerParams(needs_layout_passes=False),
)
def _gmf_join(urows_hbm, irows_hbm, params_hbm, out_hbm,
              ublk, iblk, out_v, p_v):
    wid = lax.axis_index("s") * 2 + lax.axis_index("c")
    base = wid * _BPW
    pltpu.sync_copy(params_hbm, p_v)
    w_lo = p_v[pl.ds(0, _L)]
    w_hi = p_v[pl.ds(_L, _L)]
    bias = p_v[pl.ds(2 * _L, _L)][0]
    dio = lax.iota(jnp.int32, _L)
    last_lane = dio == (_L - 1)

    def chunk(ci, carry):
        cb = pl.multiple_of(base + ci * _CH, _CH)
        pltpu.sync_copy(urows_hbm.at[pl.ds(cb, _CH), :], ublk)
        pltpu.sync_copy(irows_hbm.at[pl.ds(cb, _CH), :], iblk)

        def bbody(bi, carry):
            u_lo = ublk[bi, pl.ds(0, _L)]
            u_hi = ublk[bi, pl.ds(_L, _L)]
            i_lo = iblk[bi, pl.ds(0, _L)]
            i_hi = iblk[bi, pl.ds(_L, _L)]
            sv = u_lo * i_lo * w_lo + u_hi * i_hi * w_hi
            total = plsc.cumsum(sv) + bias
            plsc.store_scatter(out_v, [jnp.broadcast_to(ci * _CH + bi, (_L,))],
                               total, mask=last_lane)
            return carry

        return lax.fori_loop(0, _CH, bbody, carry)

    lax.fori_loop(0, _BPW // _CH, chunk, 0)
    pltpu.sync_copy(out_v, out_hbm.at[pl.ds(base, _BPW)])


def kernel(user, item, user_table, item_table, W, b):
    params = jnp.concatenate(
        [W.reshape(-1), b.reshape(-1), jnp.zeros((15,), jnp.float32)])
    urows, irows = _gmf_sweep(user.astype(jnp.int32), item.astype(jnp.int32),
                              user_table.T, item_table.T)
    return _gmf_join(urows, irows, params)


# speedup vs baseline: 1.4692x; 1.2706x over previous
import functools

import jax
import jax.numpy as jnp
from jax import lax
from jax.experimental import pallas as pl
from jax.experimental.pallas import tpu as pltpu
from jax.experimental.pallas import tpu_sc as plsc

_B = 16384
_D = 32
_L = 16
_NW = 32
_BPW = _B // _NW
_V = 1000000
_RPW = 31360
_WIN = 1024
_NWIN = 31
_SMAX = 999040
_NSTAGE = 128
_DUMP = _B
_SROWS = _B + _NSTAGE

_mesh = plsc.VectorSubcoreMesh(core_axis_name="c", subcore_axis_name="s")


@functools.partial(
    pl.kernel,
    out_type=(jax.ShapeDtypeStruct((_SROWS, 128), jnp.float32),
              jax.ShapeDtypeStruct((_SROWS, 128), jnp.float32)),
    mesh=_mesh,
    scratch_types=[
        pltpu.VMEM((_B,), jnp.int32),
        pltpu.VMEM((_B,), jnp.int32),
        pltpu.VMEM((2, _D, _WIN), jnp.float32),
        pltpu.VMEM((_NSTAGE, 128), jnp.float32),
        pltpu.VMEM((_NSTAGE,), jnp.int32),
        pltpu.SemaphoreType.DMA((2,)),
        pltpu.SemaphoreType.DMA,
    ],
    compiler_params=pltpu.CompilerParams(needs_layout_passes=False),
)
def _gmf_sweep(user_hbm, item_hbm, utab_hbm, itab_hbm, urows_hbm, irows_hbm,
               idx_v, ent, blk, srows, sidx, wsems, fsem):
    wid = lax.axis_index("s") * 2 + lax.axis_index("c")
    lo = wid * _RPW
    rlen = jnp.minimum(lo + _RPW, _V) - lo
    dio = lax.iota(jnp.int32, _L)
    ent2 = idx_v

    def win_start(j):
        return jnp.minimum(lo + j * _WIN, _SMAX)

    def fire(tab, j, slot):
        off = pl.multiple_of(win_start(j), 128)
        pltpu.async_copy(tab.at[:, pl.ds(off, _WIN)], blk.at[slot],
                         wsems.at[slot])

    def drain(tab, slot):
        pltpu.make_async_copy(
            tab.at[:, pl.ds(0, _WIN)], blk.at[slot], wsems.at[slot]).wait()

    def run_table(tab, src_idx_hbm, out_rows_hbm):
        pltpu.sync_copy(src_idx_hbm, idx_v)
        for q in range(_NSTAGE // _L):
            sidx[pl.ds(q * _L, _L)] = jnp.full((_L,), _DUMP, jnp.int32)

        def cbody(v, ptr):
            voff = pl.multiple_of(v * _L, _L)
            iv = idx_v[pl.ds(voff, _L)]
            rel = iv - lo
            m = (rel >= 0) & (rel < rlen)
            packed = rel | ((v * _L + dio) << 15)
            plsc.store_compressed(ent.at[pl.ds(ptr, _L)], packed, mask=m)
            return ptr + plsc.all_reduce_population_count(m)[0]

        nent = lax.fori_loop(0, _B // _L, cbody, jnp.int32(0))
        nv = (nent + _L - 1) // _L

        def bucket(j, ptr2):
            s_rel = win_start(j) - lo

            def qbody(q, ptr2):
                qoff = pl.multiple_of(q * _L, _L)
                ev = ent[pl.ds(qoff, _L)]
                rel_e = ev & 32767
                mm = ((q * _L + dio) < nent) & (rel_e >= s_rel) \
                    & (rel_e < s_rel + _WIN)
                plsc.store_compressed(ent2.at[pl.ds(ptr2, _L)], ev, mask=mm)
                return ptr2 + plsc.all_reduce_population_count(mm)[0]

            return lax.fori_loop(0, nv, qbody, ptr2)

        def process(j, slot, p0, p1, k):
            s_rel = win_start(j) - lo
            nch = (p1 - p0 + _L - 1) // _L

            def chunk(c, k):
                e0 = p0 + c * _L
                ev = plsc.load_gather(ent2, [e0 + dio])
                rem = jnp.minimum(p1 - e0, _L)
                m = dio < rem
                lv = jnp.clip((ev & 32767) - s_rel, 0, _WIN - 1)
                bv = ev >> 15
                kv = k + dio
                plsc.store_scatter(sidx, [kv], bv, mask=m)
                for d in range(_D):
                    vals = plsc.load_gather(
                        blk.at[slot], [jnp.broadcast_to(d, (_L,)), lv])
                    plsc.store_scatter(
                        srows, [kv, jnp.broadcast_to(d, (_L,))], vals, mask=m)
                k = k + rem

                @pl.when(k >= _NSTAGE - _L)
                def _():
                    pltpu.async_copy(srows, out_rows_hbm.at[sidx],
                                     fsem).wait()
                return lax.select(k >= _NSTAGE - _L, jnp.int32(0), k)

            return lax.fori_loop(0, nch, chunk, k)

        fire(tab, 0, 0)

        def wloop(jj, carry):
            k, ptr2 = carry
            j0 = jj * 2
            fire(tab, j0 + 1, 1)
            p1 = bucket(j0, ptr2)
            drain(tab, 0)
            k = process(j0, 0, ptr2, p1, k)
            fire(tab, j0 + 2, 0)
            p2 = bucket(j0 + 1, p1)
            drain(tab, 1)
            k = process(j0 + 1, 1, p1, p2, k)
            return k, p2

        k, ptr2 = lax.fori_loop(0, _NWIN // 2, wloop,
                                (jnp.int32(0), jnp.int32(0)))
        pl_last = bucket(_NWIN - 1, ptr2)
        drain(tab, 0)
        k = process(_NWIN - 1, 0, ptr2, pl_last, k)

        pltpu.async_copy(srows, out_rows_hbm.at[sidx], fsem).wait()

    run_table(utab_hbm, user_hbm, urows_hbm)
    run_table(itab_hbm, item_hbm, irows_hbm)


_CH = 128


@functools.partial(
    pl.kernel,
    out_type=jax.ShapeDtypeStruct((_B,), jnp.float32),
    mesh=_mesh,
    scratch_types=[
        pltpu.VMEM((_CH, 128), jnp.float32),
        pltpu.VMEM((_CH, 128), jnp.float32),
        pltpu.VMEM((_BPW,), jnp.float32),
        pltpu.VMEM((3 * _L,), jnp.float32),
    ],
    compiler_params=pltpu.CompilerParams(needs_layout_passes=False),
)
def _gmf_join(urows_hbm, irows_hbm, params_hbm, out_hbm,
              ublk, iblk, out_v, p_v):
    wid = lax.axis_index("s") * 2 + lax.axis_index("c")
    base = wid * _BPW
    pltpu.sync_copy(params_hbm, p_v)
    w_lo = p_v[pl.ds(0, _L)]
    w_hi = p_v[pl.ds(_L, _L)]
    bias = p_v[pl.ds(2 * _L, _L)][0]
    dio = lax.iota(jnp.int32, _L)
    last_lane = dio == (_L - 1)

    def chunk(ci, carry):
        cb = pl.multiple_of(base + ci * _CH, _CH)
        pltpu.sync_copy(urows_hbm.at[pl.ds(cb, _CH), :], ublk)
        pltpu.sync_copy(irows_hbm.at[pl.ds(cb, _CH), :], iblk)

        def bbody(bi, carry):
            u_lo = ublk[bi, pl.ds(0, _L)]
            u_hi = ublk[bi, pl.ds(_L, _L)]
            i_lo = iblk[bi, pl.ds(0, _L)]
            i_hi = iblk[bi, pl.ds(_L, _L)]
            sv = u_lo * i_lo * w_lo + u_hi * i_hi * w_hi
            total = plsc.cumsum(sv) + bias
            plsc.store_scatter(out_v, [jnp.broadcast_to(ci * _CH + bi, (_L,))],
                               total, mask=last_lane)
            return carry

        return lax.fori_loop(0, _CH, bbody, carry)

    lax.fori_loop(0, _BPW // _CH, chunk, 0)
    pltpu.sync_copy(out_v, out_hbm.at[pl.ds(base, _BPW)])


def kernel(user, item, user_table, item_table, W, b):
    params = jnp.concatenate(
        [W.reshape(-1), b.reshape(-1), jnp.zeros((15,), jnp.float32)])
    urows, irows = _gmf_sweep(user.astype(jnp.int32), item.astype(jnp.int32),
                              user_table.T, item_table.T)
    return _gmf_join(urows, irows, params)
